# baseline (speedup 1.0000x reference)
import jax
import jax.numpy as jnp
from jax.experimental import pallas as pl

_NSPLIT = 4


def _body(*refs):
    x_refs = refs[:_NSPLIT]
    wg_ref, bg_ref, wn_ref, bn_ref, o_ref = refs[_NSPLIT:]
    gates = [
        jax.lax.dot_general(
            wg_ref[...], xr[...], (((1,), (1,)), ((), ())),
            preferred_element_type=jnp.float32)
        for xr in x_refs
    ]
    m = gates[0].max()
    for g in gates[1:]:
        m = jnp.maximum(m, g.max())
    es = [jnp.exp(g - m) for g in gates]
    s = sum(e.sum() for e in es)
    pooled = jnp.dot(es[0], x_refs[0][...],
                     preferred_element_type=jnp.float32)
    for e, xr in zip(es[1:], x_refs[1:]):
        pooled = pooled + jnp.dot(e, xr[...],
                                  preferred_element_type=jnp.float32)
    inv = 1.0 / (s + 1e-16)
    out = jnp.dot(pooled * inv, wn_ref[...],
                  preferred_element_type=jnp.float32) + bn_ref[...] * (s * inv)
    o_ref[0] = out


def kernel(x, Wg, bg, Wn, bn):
    bz, n, f = x.shape
    xf = x.reshape(bz * n, f)
    wgT = Wg.reshape(1, f)
    bg2 = bg.reshape(1, 1)
    bn2 = bn.reshape(1, f)
    c = n // _NSPLIT
    grid = (bz,)

    def mk_idx(i):
        return lambda b: (b * _NSPLIT + i, 0)

    x_specs = [pl.BlockSpec((c, f), mk_idx(i)) for i in range(_NSPLIT)]
    return pl.pallas_call(
        _body,
        grid=grid,
        in_specs=x_specs + [
            pl.BlockSpec((1, f), lambda b: (0, 0)),
            pl.BlockSpec((1, 1), lambda b: (0, 0)),
            pl.BlockSpec((f, f), lambda b: (0, 0)),
            pl.BlockSpec((1, f), lambda b: (0, 0)),
        ],
        out_specs=pl.BlockSpec((1, 1, f), lambda b: (b, 0, 0)),
        out_shape=jax.ShapeDtypeStruct((bz, 1, f), jnp.float32),
    )(*([xf] * _NSPLIT), wgT, bg2, Wn, bn2).reshape(bz, f)
